# baseline (speedup 1.0000x reference)
import jax
import jax.numpy as jnp
from jax.experimental import pallas as pl
from jax.experimental.pallas import tpu as pltpu

B, L, D = 16, 4096, 1024
NL = 8
LBLK = L // NL


def _body(mask_ref, feats_ref, out_ref, amax_ref, asum_ref, acnt_ref):
    l = pl.program_id(1)
    x = feats_ref[0]
    mrow = mask_ref[0]
    ones = jnp.ones((1, 1), jnp.float32)
    mcol = jax.lax.dot_general(
        mrow, ones, (((0,), (0,)), ((), ())),
        preferred_element_type=jnp.float32)
    mb = mcol > 0.5
    bmax = jnp.max(jnp.where(mb, x, jnp.float32(-1e30)), axis=0,
                   keepdims=True)
    bsum = jnp.sum(jnp.where(mb, x, 0.0), axis=0, keepdims=True)
    bcnt = jnp.sum(mrow)

    @pl.when(l == 0)
    def _():
        amax_ref[...] = bmax
        asum_ref[...] = bsum
        acnt_ref[0] = bcnt

    @pl.when(l > 0)
    def _():
        amax_ref[...] = jnp.maximum(amax_ref[...], bmax)
        asum_ref[...] = asum_ref[...] + bsum
        acnt_ref[0] = acnt_ref[0] + bcnt

    @pl.when(l == NL - 1)
    def _():
        out_ref[0, 0, :D] = amax_ref[0]
        out_ref[0, 0, D:] = asum_ref[0] / acnt_ref[0]


def kernel(feats, mask):
    maskf = mask.astype(jnp.float32).reshape(B, 1, L)
    out = pl.pallas_call(
        _body,
        grid=(B, NL),
        in_specs=[
            pl.BlockSpec((1, 1, LBLK), lambda b, l: (b, 0, l)),
            pl.BlockSpec((1, LBLK, D), lambda b, l: (b, l, 0)),
        ],
        out_specs=pl.BlockSpec((1, 1, 2 * D), lambda b, l: (b, 0, 0)),
        out_shape=jax.ShapeDtypeStruct((B, 1, 2 * D), jnp.float32),
        scratch_shapes=[
            pltpu.VMEM((1, D), jnp.float32),
            pltpu.VMEM((1, D), jnp.float32),
            pltpu.SMEM((1,), jnp.float32),
        ],
    )(maskf, feats)
    return out.reshape(B, 2 * D)
